# Initial kernel scaffold; baseline (speedup 1.0000x reference)
import jax
import jax.numpy as jnp
from jax.experimental import pallas as pl


def kernel(batch_features, edge_indexes, i_params, a_params, norm1_g, norm1_b, norm2_g, norm2_b):
    raise NotImplementedError("write your pallas kernel here")


# speedup vs baseline: 8.1100x; 8.1100x over previous
import functools

import jax
import jax.numpy as jnp
from jax.experimental import pallas as pl
from jax.experimental.pallas import tpu as pltpu

AUDIO_LEN = 48
TOTAL_LEN = 512
IMG_LEN = TOTAL_LEN - AUDIO_LEN
D = 192
EPG = 2048


def _gat_block(h, adj, W, a_src, a_dst, W_out, b_out):
    z = jnp.dot(h, W, preferred_element_type=jnp.float32)
    es = jnp.sum(z * a_src, axis=1, keepdims=True)
    ed = jnp.sum(z * a_dst, axis=1, keepdims=True)
    e = ed + es.T
    e = jnp.where(e >= 0.0, e, 0.2 * e)
    e = jnp.where(adj, e, jnp.float32(-1e9))
    e = e - jnp.max(e, axis=1, keepdims=True)
    p = jnp.exp(e)
    alpha = p / jnp.sum(p, axis=1, keepdims=True)
    msg = jnp.dot(alpha, z, preferred_element_type=jnp.float32)
    has = jnp.any(adj, axis=1, keepdims=True)
    msg = jnp.where(has, msg, 0.0)
    g = jnp.where(msg > 0.0, msg, jnp.exp(jnp.minimum(msg, 0.0)) - 1.0)
    return jnp.dot(g, W_out, preferred_element_type=jnp.float32) + b_out


def _ln_rows(x, g, b):
    m = jnp.mean(x, axis=1, keepdims=True)
    xc = x - m
    v = jnp.mean(xc * xc, axis=1, keepdims=True)
    return xc * jax.lax.rsqrt(v + 1e-5) * g + b


def _han_kernel(bf_ref, src_ref, dst_ref, wmats_ref, vecs_ref, out_ref):
    img = bf_ref[0, :IMG_LEN, :]
    aud = bf_ref[0, IMG_LEN:, :]
    src = src_ref[0]
    dst = dst_ref[0]

    i2a = (src < IMG_LEN) & (dst >= IMG_LEN)
    a2i = (src >= IMG_LEN) & (dst < IMG_LEN)

    img_iota = jax.lax.broadcasted_iota(jnp.int32, (IMG_LEN, EPG), 0)
    aud_iota = jax.lax.broadcasted_iota(jnp.int32, (AUDIO_LEN, EPG), 0)

    s_img = ((img_iota == src) & i2a).astype(jnp.float32)
    d_aud = ((aud_iota == dst - IMG_LEN)).astype(jnp.float32)
    U = jax.lax.dot_general(s_img, d_aud, (((1,), (1,)), ((), ())),
                            preferred_element_type=jnp.float32)
    sent = ((jax.lax.broadcasted_iota(jnp.int32, (IMG_LEN, AUDIO_LEN), 0) == IMG_LEN - 1)
            & (jax.lax.broadcasted_iota(jnp.int32, (IMG_LEN, AUDIO_LEN), 1) == AUDIO_LEN - 1))
    Ub = ((U > 0.0) | sent).astype(jnp.float32)

    s_aud = ((aud_iota == src - IMG_LEN) & a2i).astype(jnp.float32)
    d_img = (img_iota == dst).astype(jnp.float32)
    V = jax.lax.dot_general(s_aud, d_img, (((1,), (1,)), ((), ())),
                            preferred_element_type=jnp.float32)
    Vb = (V > 0.0).astype(jnp.float32)

    adj_img = jax.lax.dot_general(Vb, Ub, (((0,), (1,)), ((), ())),
                                  preferred_element_type=jnp.float32) > 0.0
    adj_aud = jax.lax.dot_general(Ub, Vb, (((0,), (1,)), ((), ())),
                                  preferred_element_type=jnp.float32) > 0.0

    w = wmats_ref[...]
    v = vecs_ref[...]
    out_i = _gat_block(img, adj_img, w[0], v[0:1], v[1:2], w[1], v[2:3])
    out_a = _gat_block(aud, adj_aud, w[2], v[3:4], v[4:5], w[3], v[5:6])

    out_ref[0, :IMG_LEN, :] = _ln_rows(out_i, v[6:7], v[7:8])
    out_ref[0, IMG_LEN:, :] = _ln_rows(out_a, v[8:9], v[9:10])


@jax.jit
def kernel(batch_features, edge_indexes, i_params, a_params, norm1_g, norm1_b, norm2_g, norm2_b):
    Bn = batch_features.shape[0]
    src = edge_indexes[1].reshape(Bn, 1, EPG).astype(jnp.int32)
    dst = edge_indexes[0].reshape(Bn, 1, EPG).astype(jnp.int32)

    wmats = jnp.stack([i_params['W'], i_params['W_out'],
                       a_params['W'], a_params['W_out']])
    vecs = jnp.stack([i_params['a_src'], i_params['a_dst'], i_params['b_out'],
                      a_params['a_src'], a_params['a_dst'], a_params['b_out'],
                      norm1_g, norm1_b, norm2_g, norm2_b])

    return pl.pallas_call(
        _han_kernel,
        grid=(Bn,),
        in_specs=[
            pl.BlockSpec((1, TOTAL_LEN, D), lambda b: (b, 0, 0)),
            pl.BlockSpec((1, 1, EPG), lambda b: (b, 0, 0)),
            pl.BlockSpec((1, 1, EPG), lambda b: (b, 0, 0)),
            pl.BlockSpec((4, D, D), lambda b: (0, 0, 0)),
            pl.BlockSpec((10, D), lambda b: (0, 0)),
        ],
        out_specs=pl.BlockSpec((1, TOTAL_LEN, D), lambda b: (b, 0, 0)),
        out_shape=jax.ShapeDtypeStruct((Bn, TOTAL_LEN, D), jnp.float32),
        compiler_params=pltpu.CompilerParams(
            dimension_semantics=("parallel",)),
    )(batch_features, src, dst, wmats, vecs)


# speedup vs baseline: 8.3879x; 1.0343x over previous
import functools

import jax
import jax.numpy as jnp
from jax import lax
from jax.experimental import pallas as pl
from jax.experimental.pallas import tpu as pltpu
from jax.experimental.pallas import tpu_sc as plsc

AUDIO_LEN = 48
TOTAL_LEN = 512
IMG_LEN = TOTAL_LEN - AUDIO_LEN
D = 192
EPG = 2048
UV_WORDS = IMG_LEN * AUDIO_LEN
LANES = 16


def _sc_build_adj(src_hbm, dst_hbm, u_hbm, v_hbm, src_v, dst_v, u_v, v_v):
    wid = lax.axis_index("c") * 16 + lax.axis_index("s")

    pltpu.sync_copy(src_hbm.at[wid], src_v)
    pltpu.sync_copy(dst_hbm.at[wid], dst_v)

    zeros = jnp.zeros((LANES,), jnp.float32)

    def zero_body(i, _):
        u_v[pl.ds(i * LANES, LANES)] = zeros
        v_v[pl.ds(i * LANES, LANES)] = zeros
        return 0

    lax.fori_loop(0, UV_WORDS // LANES, zero_body, 0)

    ones = jnp.ones((LANES,), jnp.float32)

    def edge_body(i, _):
        s = src_v[pl.ds(i * LANES, LANES)]
        d = dst_v[pl.ds(i * LANES, LANES)]
        i2a = (s < IMG_LEN) & (d >= IMG_LEN)
        a2i = (s >= IMG_LEN) & (d < IMG_LEN)
        u_idx = jnp.where(i2a, s * AUDIO_LEN + (d - IMG_LEN), 0)
        v_idx = jnp.where(a2i, (s - IMG_LEN) * IMG_LEN + d, 0)
        plsc.store_scatter(u_v, [u_idx], ones, mask=i2a)
        plsc.store_scatter(v_v, [v_idx], ones, mask=a2i)
        return 0

    lax.fori_loop(0, EPG // LANES, edge_body, 0)

    pltpu.sync_copy(u_v, u_hbm.at[wid])
    pltpu.sync_copy(v_v, v_hbm.at[wid])


def _gat_block(h, adj, W, a_src, a_dst, W_out, b_out):
    z = jnp.dot(h, W, preferred_element_type=jnp.float32)
    es = jnp.sum(z * a_src, axis=1, keepdims=True)
    ed = jnp.sum(z * a_dst, axis=1, keepdims=True)
    e = ed + es.T
    e = jnp.where(e >= 0.0, e, 0.2 * e)
    e = jnp.where(adj, e, jnp.float32(-1e9))
    e = e - jnp.max(e, axis=1, keepdims=True)
    p = jnp.exp(e)
    alpha = p / jnp.sum(p, axis=1, keepdims=True)
    msg = jnp.dot(alpha, z, preferred_element_type=jnp.float32)
    has = jnp.any(adj, axis=1, keepdims=True)
    msg = jnp.where(has, msg, 0.0)
    g = jnp.where(msg > 0.0, msg, jnp.exp(jnp.minimum(msg, 0.0)) - 1.0)
    return jnp.dot(g, W_out, preferred_element_type=jnp.float32) + b_out


def _ln_rows(x, g, b):
    m = jnp.mean(x, axis=1, keepdims=True)
    xc = x - m
    v = jnp.mean(xc * xc, axis=1, keepdims=True)
    return xc * lax.rsqrt(v + 1e-5) * g + b


def _han_kernel(bf_ref, u_ref, v_ref, wmats_ref, vecs_ref, out_ref):
    img = bf_ref[0, :IMG_LEN, :]
    aud = bf_ref[0, IMG_LEN:, :]

    U = u_ref[0]
    V = v_ref[0]
    sent = ((lax.broadcasted_iota(jnp.int32, (IMG_LEN, AUDIO_LEN), 0) == IMG_LEN - 1)
            & (lax.broadcasted_iota(jnp.int32, (IMG_LEN, AUDIO_LEN), 1) == AUDIO_LEN - 1))
    Ub = jnp.maximum(U, sent.astype(jnp.float32))

    adj_img = lax.dot_general(V, Ub, (((0,), (1,)), ((), ())),
                              preferred_element_type=jnp.float32) > 0.0
    adj_aud = lax.dot_general(Ub, V, (((0,), (1,)), ((), ())),
                              preferred_element_type=jnp.float32) > 0.0

    w = wmats_ref[...]
    v = vecs_ref[...]
    out_i = _gat_block(img, adj_img, w[0], v[0:1], v[1:2], w[1], v[2:3])
    out_a = _gat_block(aud, adj_aud, w[2], v[3:4], v[4:5], w[3], v[5:6])

    out_ref[0, :IMG_LEN, :] = _ln_rows(out_i, v[6:7], v[7:8])
    out_ref[0, IMG_LEN:, :] = _ln_rows(out_a, v[8:9], v[9:10])


@jax.jit
def kernel(batch_features, edge_indexes, i_params, a_params, norm1_g, norm1_b, norm2_g, norm2_b):
    Bn = batch_features.shape[0]
    src = edge_indexes[1].reshape(Bn, EPG).astype(jnp.int32)
    dst = edge_indexes[0].reshape(Bn, EPG).astype(jnp.int32)

    sc_build = functools.partial(
        pl.kernel,
        mesh=plsc.VectorSubcoreMesh(core_axis_name="c", subcore_axis_name="s"),
        out_type=[
            jax.ShapeDtypeStruct((Bn, UV_WORDS), jnp.float32),
            jax.ShapeDtypeStruct((Bn, UV_WORDS), jnp.float32),
        ],
        scratch_types=[
            pltpu.VMEM((EPG,), jnp.int32),
            pltpu.VMEM((EPG,), jnp.int32),
            pltpu.VMEM((UV_WORDS,), jnp.float32),
            pltpu.VMEM((UV_WORDS,), jnp.float32),
        ],
        compiler_params=pltpu.CompilerParams(needs_layout_passes=False),
    )(_sc_build_adj)
    u_flat, v_flat = sc_build(src, dst)
    u = u_flat.reshape(Bn, IMG_LEN, AUDIO_LEN)
    v = v_flat.reshape(Bn, AUDIO_LEN, IMG_LEN)

    wmats = jnp.stack([i_params['W'], i_params['W_out'],
                       a_params['W'], a_params['W_out']])
    vecs = jnp.stack([i_params['a_src'], i_params['a_dst'], i_params['b_out'],
                      a_params['a_src'], a_params['a_dst'], a_params['b_out'],
                      norm1_g, norm1_b, norm2_g, norm2_b])

    return pl.pallas_call(
        _han_kernel,
        grid=(Bn,),
        in_specs=[
            pl.BlockSpec((1, TOTAL_LEN, D), lambda b: (b, 0, 0)),
            pl.BlockSpec((1, IMG_LEN, AUDIO_LEN), lambda b: (b, 0, 0)),
            pl.BlockSpec((1, AUDIO_LEN, IMG_LEN), lambda b: (b, 0, 0)),
            pl.BlockSpec((4, D, D), lambda b: (0, 0, 0)),
            pl.BlockSpec((10, D), lambda b: (0, 0)),
        ],
        out_specs=pl.BlockSpec((1, TOTAL_LEN, D), lambda b: (b, 0, 0)),
        out_shape=jax.ShapeDtypeStruct((Bn, TOTAL_LEN, D), jnp.float32),
        compiler_params=pltpu.CompilerParams(
            dimension_semantics=("parallel",)),
    )(batch_features, u, v, wmats, vecs)


# speedup vs baseline: 8.5307x; 1.0170x over previous
import functools

import jax
import jax.numpy as jnp
from jax import lax
from jax.experimental import pallas as pl
from jax.experimental.pallas import tpu as pltpu
from jax.experimental.pallas import tpu_sc as plsc

AUDIO_LEN = 48
TOTAL_LEN = 512
IMG_LEN = TOTAL_LEN - AUDIO_LEN
D = 192
EPG = 2048
UV_WORDS = IMG_LEN * AUDIO_LEN
LANES = 16


def _sc_build_adj(src_hbm, dst_hbm, u_hbm, v_hbm, src_v, dst_v, u_v, v_v):
    wid = lax.axis_index("c") * 16 + lax.axis_index("s")

    pltpu.sync_copy(src_hbm.at[wid], src_v)
    pltpu.sync_copy(dst_hbm.at[wid], dst_v)

    zeros = jnp.zeros((LANES,), jnp.float32)

    def zero_body(i, _):
        u_v[pl.ds(i * LANES, LANES)] = zeros
        v_v[pl.ds(i * LANES, LANES)] = zeros
        return 0

    lax.fori_loop(0, UV_WORDS // LANES, zero_body, 0)

    ones = jnp.ones((LANES,), jnp.float32)

    def edge_body(i, _):
        s = src_v[pl.ds(i * LANES, LANES)]
        d = dst_v[pl.ds(i * LANES, LANES)]
        i2a = (s < IMG_LEN) & (d >= IMG_LEN)
        a2i = (s >= IMG_LEN) & (d < IMG_LEN)
        u_idx = jnp.where(i2a, s * AUDIO_LEN + (d - IMG_LEN), 0)
        v_idx = jnp.where(a2i, (s - IMG_LEN) * IMG_LEN + d, 0)
        plsc.store_scatter(u_v, [u_idx], ones, mask=i2a)
        plsc.store_scatter(v_v, [v_idx], ones, mask=a2i)
        return 0

    lax.fori_loop(0, EPG // LANES, edge_body, 0)

    pltpu.sync_copy(u_v, u_hbm.at[wid])
    pltpu.sync_copy(v_v, v_hbm.at[wid])


def _gat_block(h, adj, W, a_src, a_dst, W_out, b_out):
    z = jnp.dot(h.astype(jnp.bfloat16), W.astype(jnp.bfloat16),
                preferred_element_type=jnp.float32)
    es = jnp.sum(z * a_src, axis=1, keepdims=True)
    ed = jnp.sum(z * a_dst, axis=1, keepdims=True)
    e = ed + es.T
    e = jnp.where(e >= 0.0, e, 0.2 * e)
    e = jnp.where(adj, e, jnp.float32(-1e9))
    e = e - jnp.max(e, axis=1, keepdims=True)
    p = jnp.exp(e)
    alpha = p / jnp.sum(p, axis=1, keepdims=True)
    msg = jnp.dot(alpha.astype(jnp.bfloat16), z.astype(jnp.bfloat16),
                  preferred_element_type=jnp.float32)
    has = jnp.any(adj, axis=1, keepdims=True)
    msg = jnp.where(has, msg, 0.0)
    g = jnp.where(msg > 0.0, msg, jnp.exp(jnp.minimum(msg, 0.0)) - 1.0)
    return jnp.dot(g.astype(jnp.bfloat16), W_out.astype(jnp.bfloat16),
                   preferred_element_type=jnp.float32) + b_out


def _ln_rows(x, g, b):
    m = jnp.mean(x, axis=1, keepdims=True)
    xc = x - m
    v = jnp.mean(xc * xc, axis=1, keepdims=True)
    return xc * lax.rsqrt(v + 1e-5) * g + b


def _han_kernel(bf_ref, u_ref, v_ref, wmats_ref, vecs_ref, out_ref):
    img = bf_ref[0, :IMG_LEN, :]
    aud = bf_ref[0, IMG_LEN:, :]

    U = u_ref[0]
    V = v_ref[0]
    sent = ((lax.broadcasted_iota(jnp.int32, (IMG_LEN, AUDIO_LEN), 0) == IMG_LEN - 1)
            & (lax.broadcasted_iota(jnp.int32, (IMG_LEN, AUDIO_LEN), 1) == AUDIO_LEN - 1))
    Ub = jnp.maximum(U, sent.astype(jnp.float32)).astype(jnp.bfloat16)
    Vh = V.astype(jnp.bfloat16)

    adj_img = lax.dot_general(Vh, Ub, (((0,), (1,)), ((), ())),
                              preferred_element_type=jnp.float32) > 0.0
    adj_aud = lax.dot_general(Ub, Vh, (((0,), (1,)), ((), ())),
                              preferred_element_type=jnp.float32) > 0.0

    w = wmats_ref[...]
    v = vecs_ref[...]
    out_i = _gat_block(img, adj_img, w[0], v[0:1], v[1:2], w[1], v[2:3])
    out_a = _gat_block(aud, adj_aud, w[2], v[3:4], v[4:5], w[3], v[5:6])

    out_ref[0, :IMG_LEN, :] = _ln_rows(out_i, v[6:7], v[7:8])
    out_ref[0, IMG_LEN:, :] = _ln_rows(out_a, v[8:9], v[9:10])


@jax.jit
def kernel(batch_features, edge_indexes, i_params, a_params, norm1_g, norm1_b, norm2_g, norm2_b):
    Bn = batch_features.shape[0]
    src = edge_indexes[1].reshape(Bn, EPG).astype(jnp.int32)
    dst = edge_indexes[0].reshape(Bn, EPG).astype(jnp.int32)

    sc_build = functools.partial(
        pl.kernel,
        mesh=plsc.VectorSubcoreMesh(core_axis_name="c", subcore_axis_name="s"),
        out_type=[
            jax.ShapeDtypeStruct((Bn, UV_WORDS), jnp.float32),
            jax.ShapeDtypeStruct((Bn, UV_WORDS), jnp.float32),
        ],
        scratch_types=[
            pltpu.VMEM((EPG,), jnp.int32),
            pltpu.VMEM((EPG,), jnp.int32),
            pltpu.VMEM((UV_WORDS,), jnp.float32),
            pltpu.VMEM((UV_WORDS,), jnp.float32),
        ],
        compiler_params=pltpu.CompilerParams(needs_layout_passes=False),
    )(_sc_build_adj)
    u_flat, v_flat = sc_build(src, dst)
    u = u_flat.reshape(Bn, IMG_LEN, AUDIO_LEN)
    v = v_flat.reshape(Bn, AUDIO_LEN, IMG_LEN)

    wmats = jnp.stack([i_params['W'], i_params['W_out'],
                       a_params['W'], a_params['W_out']])
    vecs = jnp.stack([i_params['a_src'], i_params['a_dst'], i_params['b_out'],
                      a_params['a_src'], a_params['a_dst'], a_params['b_out'],
                      norm1_g, norm1_b, norm2_g, norm2_b])

    return pl.pallas_call(
        _han_kernel,
        grid=(Bn,),
        in_specs=[
            pl.BlockSpec((1, TOTAL_LEN, D), lambda b: (b, 0, 0)),
            pl.BlockSpec((1, IMG_LEN, AUDIO_LEN), lambda b: (b, 0, 0)),
            pl.BlockSpec((1, AUDIO_LEN, IMG_LEN), lambda b: (b, 0, 0)),
            pl.BlockSpec((4, D, D), lambda b: (0, 0, 0)),
            pl.BlockSpec((10, D), lambda b: (0, 0)),
        ],
        out_specs=pl.BlockSpec((1, TOTAL_LEN, D), lambda b: (b, 0, 0)),
        out_shape=jax.ShapeDtypeStruct((Bn, TOTAL_LEN, D), jnp.float32),
        compiler_params=pltpu.CompilerParams(
            dimension_semantics=("parallel",)),
    )(batch_features, u, v, wmats, vecs)
